# baseline (speedup 1.0000x reference)
import numpy as np
import jax
import jax.numpy as jnp
from jax import lax
from jax.experimental import pallas as pl
from jax.experimental.pallas import tpu as pltpu

L = 8192
NBINS = 180
NPAD = 256
K = 50
KPAD = 64
ROWS = 128

_n = np.arange(L)
_k = np.arange(NBINS)
_ang = (2.0 * np.pi / L) * ((_k[:, None] * _n[None, :]) % L).astype(np.float64)
_TRIG_T = np.concatenate([np.cos(_ang), np.sin(_ang)], axis=0).T.astype(np.float32)
_TRIG_T = np.ascontiguousarray(_TRIG_T)


NCHUNK = 8
CHUNK = L // NCHUNK


def _body(x_ref, trig_ref, w1t_ref, b1_ref, w2t_ref, b2_ref, out_ref, acc_ref):
    i = pl.program_id(0)

    @pl.when(i == 0)
    def _init():
        acc_ref[...] = jnp.zeros((ROWS, 2 * NBINS), jnp.float32)

    acc_ref[...] += jnp.dot(x_ref[...], trig_ref[...],
                            preferred_element_type=jnp.float32,
                            precision=lax.Precision.HIGHEST)

    @pl.when(i == NCHUNK - 1)
    def _finish():
        _tail(acc_ref, w1t_ref, b1_ref, w2t_ref, b2_ref, out_ref)


def _tail(acc_ref, w1t_ref, b1_ref, w2t_ref, b2_ref, out_ref):
    res = acc_ref[...]
    re = res[:, :NBINS]
    im = res[:, NBINS:]
    mag2 = re * re + im * im

    col = lax.broadcasted_iota(jnp.int32, (ROWS, NPAD), 1)
    mag2p = jnp.concatenate(
        [mag2, jnp.full((ROWS, NPAD - NBINS), -1.0, jnp.float32)], axis=1)
    mag2p = jnp.where(col == 0, -1.0, mag2p)

    RC = 8
    jlt = lax.broadcasted_iota(jnp.int32, (1, NPAD, NPAD), 2) \
        < lax.broadcasted_iota(jnp.int32, (1, NPAD, NPAD), 1)
    targets = ((NPAD - K) + lax.broadcasted_iota(jnp.int32, (1, 1, KPAD), 2)
               ).astype(jnp.float32)
    i_f = lax.broadcasted_iota(jnp.int32, (RC, NPAD, KPAD), 1).astype(jnp.float32)

    chunks = []
    for ci in range(ROWS // RC):
        a = mag2p[ci * RC:(ci + 1) * RC, :]
        ai = a[:, :, None]
        aj = a[:, None, :]
        below = jnp.where((aj < ai) | ((aj == ai) & jlt), 1.0, 0.0)
        rank = jnp.sum(below, axis=2)
        sel = jnp.where(rank[:, :, None] == targets, i_f, 0.0)
        chunks.append(jnp.sum(sel, axis=1))
    outk = jnp.concatenate(chunks, axis=0)
    x50 = outk[:, :K]

    l1 = jnp.dot(x50, w1t_ref[...], preferred_element_type=jnp.float32,
                 precision=lax.Precision.HIGHEST) + b1_ref[...]
    l2 = jnp.dot(l1, w2t_ref[...], preferred_element_type=jnp.float32,
                 precision=lax.Precision.HIGHEST) + b2_ref[...]
    out_ref[...] = l2


def kernel(ple_input, W1, b1, W2, b2):
    x = ple_input.reshape(ROWS, L)
    trig = jnp.asarray(_TRIG_T)
    return pl.pallas_call(
        _body,
        grid=(NCHUNK,),
        in_specs=[
            pl.BlockSpec((ROWS, CHUNK), lambda i: (0, i)),
            pl.BlockSpec((CHUNK, 2 * NBINS), lambda i: (i, 0)),
            pl.BlockSpec((K, 70), lambda i: (0, 0)),
            pl.BlockSpec((1, 70), lambda i: (0, 0)),
            pl.BlockSpec((70, 90), lambda i: (0, 0)),
            pl.BlockSpec((1, 90), lambda i: (0, 0)),
        ],
        out_specs=pl.BlockSpec((ROWS, 90), lambda i: (0, 0)),
        scratch_shapes=[pltpu.VMEM((ROWS, 2 * NBINS), jnp.float32)],
        out_shape=jax.ShapeDtypeStruct((ROWS, 90), jnp.float32),
    )(x, trig, W1.T, b1.reshape(1, 70), W2.T, b2.reshape(1, 90))
